# baseline (speedup 1.0000x reference)
import jax
import jax.numpy as jnp
from jax.experimental import pallas as pl
from jax.experimental.pallas import tpu as pltpu

EPS_DICE = 0.0001


def _dice_block_kernel(out_ref, tgt_ref, acc_ref):
    j = pl.program_id(1)
    x = out_ref[0]
    t = tgt_ref[0]
    th = jnp.tanh(x * 0.5)
    cls = jax.lax.broadcasted_iota(jnp.int32, x.shape, 0)
    mask = t[None, :, :] == cls
    t_tot = jnp.sum(th, axis=(1, 2))
    t_int = jnp.sum(jnp.where(mask, th, 0.0), axis=(1, 2))
    cnt = jnp.sum(mask.astype(jnp.float32), axis=(1, 2))
    part = jnp.concatenate([t_tot, t_int, cnt])

    @pl.when(j == 0)
    def _init():
        acc_ref[0, 0] = part

    @pl.when(j > 0)
    def _acc():
        acc_ref[0, 0] += part


@jax.jit
def kernel(output, target):
    b, c, h, w = output.shape
    hsplit = 4
    hb = h // hsplit
    tgt = target.astype(jnp.int32)
    acc = pl.pallas_call(
        _dice_block_kernel,
        grid=(b, hsplit),
        in_specs=[
            pl.BlockSpec((1, c, hb, w), lambda i, j: (i, 0, j, 0)),
            pl.BlockSpec((1, hb, w), lambda i, j: (i, j, 0)),
        ],
        out_specs=pl.BlockSpec((1, 1, 3 * c), lambda i, j: (i, 0, 0)),
        out_shape=jax.ShapeDtypeStruct((b, 1, 3 * c), jnp.float32),
        compiler_params=pltpu.CompilerParams(
            dimension_semantics=("parallel", "arbitrary"),
        ),
    )(output, tgt)
    t_tot = acc[:, 0, :c]
    t_int = acc[:, 0, c : 2 * c]
    cnt = acc[:, 0, 2 * c :]
    hw = jnp.float32(h * w)
    sig_sum = 0.5 * t_tot + 0.5 * hw
    inter = 0.5 * t_int + 0.5 * cnt
    loss_per_channel = 1.0 - 2.0 * inter / (sig_sum + cnt + EPS_DICE)
    return loss_per_channel.sum(axis=1) / c


# speedup vs baseline: 1.4337x; 1.4337x over previous
import jax
import jax.numpy as jnp
from jax.experimental import pallas as pl
from jax.experimental.pallas import tpu as pltpu

EPS_DICE = 0.0001


def _dice_block_kernel(out_ref, tgt_ref, acc_ref):
    x = out_ref[0]
    t = tgt_ref[0]
    th = jnp.tanh(x * 0.5)
    cls = jax.lax.broadcasted_iota(jnp.int32, x.shape, 0)
    mask = t[None, :, :] == cls
    t_tot = jnp.sum(th, axis=(1, 2))
    t_int = jnp.sum(jnp.where(mask, th, 0.0), axis=(1, 2))
    cnt = jnp.sum(mask.astype(jnp.float32), axis=(1, 2))
    acc_ref[0, 0] = jnp.concatenate([t_tot, t_int, cnt])


@jax.jit
def kernel(output, target):
    b, c, h, w = output.shape
    tgt = target.astype(jnp.int32)
    acc = pl.pallas_call(
        _dice_block_kernel,
        grid=(b,),
        in_specs=[
            pl.BlockSpec((1, c, h, w), lambda i: (i, 0, 0, 0)),
            pl.BlockSpec((1, h, w), lambda i: (i, 0, 0)),
        ],
        out_specs=pl.BlockSpec((1, 1, 3 * c), lambda i: (i, 0, 0)),
        out_shape=jax.ShapeDtypeStruct((b, 1, 3 * c), jnp.float32),
        compiler_params=pltpu.CompilerParams(
            dimension_semantics=("arbitrary",),
        ),
    )(output, tgt)
    t_tot = acc[:, 0, :c]
    t_int = acc[:, 0, c : 2 * c]
    cnt = acc[:, 0, 2 * c :]
    hw = jnp.float32(h * w)
    sig_sum = 0.5 * t_tot + 0.5 * hw
    inter = 0.5 * t_int + 0.5 * cnt
    loss_per_channel = 1.0 - 2.0 * inter / (sig_sum + cnt + EPS_DICE)
    return loss_per_channel.sum(axis=1) / c
